# Initial kernel scaffold; baseline (speedup 1.0000x reference)
import jax
import jax.numpy as jnp
from jax.experimental import pallas as pl


def kernel(x, edge_index, W1, b1, W2, b2):
    raise NotImplementedError("write your pallas kernel here")


# speedup vs baseline: 14.6409x; 14.6409x over previous
import functools

import jax
import jax.numpy as jnp
from jax import lax
from jax.experimental import pallas as pl
from jax.experimental.pallas import tpu as pltpu
from jax.experimental.pallas import tpu_sc as plsc

CH = 125
HALF = 128


DEGW = 128


def _sc_degree(dst16, zeros_w, ones_w, n_pad):
    nc = dst16.shape[1]
    rpt = n_pad // 16
    mesh = plsc.VectorSubcoreMesh(core_axis_name="c", subcore_axis_name="s", num_cores=2, num_subcores=16)

    @functools.partial(
        pl.kernel, mesh=mesh,
        out_type=jax.ShapeDtypeStruct((n_pad, DEGW), jnp.float32),
        scratch_types=[
            pltpu.VMEM((nc, CH), jnp.int32),
            pltpu.VMEM((CH, DEGW), jnp.float32),
            pltpu.VMEM_SHARED((n_pad, DEGW), jnp.float32),
        ],
    )
    def deg_kernel(dst_hbm, zero_hbm, one_hbm, out_hbm, idx_v, ones_v, acc):
        c = lax.axis_index("c")
        s = lax.axis_index("s")
        base = s * rpt
        pltpu.sync_copy(dst_hbm.at[s], idx_v)
        pltpu.sync_copy(one_hbm, ones_v)
        pltpu.sync_copy(zero_hbm.at[pl.ds(base, rpt)], acc.at[pl.ds(base, rpt)])
        plsc.subcore_barrier()

        def body(j, carry):
            pltpu.sync_copy(ones_v, acc.at[idx_v.at[j]], add=True)
            return carry

        lax.fori_loop(0, nc, body, 0)
        plsc.subcore_barrier()

        @pl.when(c == 0)
        def _():
            pltpu.sync_copy(acc.at[pl.ds(base, rpt)],
                            out_hbm.at[pl.ds(base, rpt)])

    return deg_kernel(dst16, zeros_w, ones_w)


def _sc_scatter(ua, ub, src16, dst16, n_pad):
    nc = src16.shape[1]
    rpt = n_pad // 16
    mesh = plsc.VectorSubcoreMesh(core_axis_name="c", subcore_axis_name="s", num_cores=2, num_subcores=16)

    @functools.partial(
        pl.kernel, mesh=mesh,
        out_type=[jax.ShapeDtypeStruct((n_pad, HALF), jnp.float32),
                  jax.ShapeDtypeStruct((n_pad, HALF), jnp.float32)],
        scratch_types=[
            pltpu.VMEM((nc, CH), jnp.int32),
            pltpu.VMEM((nc, CH), jnp.int32),
            pltpu.VMEM((CH, HALF), jnp.float32),
            pltpu.VMEM_SHARED((n_pad, HALF), jnp.float32),
            pltpu.SemaphoreType.DMA,
        ],
    )
    def scat_kernel(ua_hbm, ub_hbm, src_hbm, dst_hbm, sa_hbm, sb_hbm,
                    isrc, idst, rows, acc, sem):
        c = lax.axis_index("c")
        s = lax.axis_index("s")
        base = s * rpt
        pltpu.sync_copy(src_hbm.at[s], isrc)
        pltpu.sync_copy(dst_hbm.at[s], idst)

        def run(u_hbm, out_hbm):
            pltpu.sync_copy(u_hbm.at[pl.ds(base, rpt)],
                            acc.at[pl.ds(base, rpt)])
            plsc.subcore_barrier()

            def body(j, carry):
                pltpu.async_copy(u_hbm.at[isrc.at[j]], rows, sem).wait()
                pltpu.sync_copy(rows, acc.at[idst.at[j]], add=True)
                return carry

            lax.fori_loop(0, nc, body, 0)
            plsc.subcore_barrier()
            pltpu.sync_copy(acc.at[pl.ds(base, rpt)],
                            out_hbm.at[pl.ds(base, rpt)])

        @pl.when(c == 0)
        def _():
            run(ua_hbm, sa_hbm)

        @pl.when(c == 1)
        def _():
            run(ub_hbm, sb_hbm)

    return scat_kernel(ua, ub, src16, dst16)


def _tc_scale_x(x_p, d0, n_pad, nin, blk):
    def body(x_ref, d0_ref, ua_ref, ub_ref, dv_ref):
        deg = d0_ref[:, :1] + 1.0
        dv = lax.rsqrt(jnp.maximum(deg, 1e-12))
        u = x_ref[...] * dv
        ua_ref[...] = u[:, :HALF]
        ub_ref[...] = u[:, HALF:]
        dv_ref[...] = dv

    grid = (n_pad // blk,)
    return pl.pallas_call(
        body,
        grid=grid,
        in_specs=[
            pl.BlockSpec((blk, nin), lambda i: (i, 0)),
            pl.BlockSpec((blk, DEGW), lambda i: (i, 0)),
        ],
        out_specs=[
            pl.BlockSpec((blk, HALF), lambda i: (i, 0)),
            pl.BlockSpec((blk, HALF), lambda i: (i, 0)),
            pl.BlockSpec((blk, 1), lambda i: (i, 0)),
        ],
        out_shape=[
            jax.ShapeDtypeStruct((n_pad, HALF), jnp.float32),
            jax.ShapeDtypeStruct((n_pad, HALF), jnp.float32),
            jax.ShapeDtypeStruct((n_pad, 1), jnp.float32),
        ],
    )(x_p, d0)


def _tc_mlp(sa, sb, dinv, W1, b1r, W2, n_pad, nin, hid, nout, blk):
    def body(sa_ref, sb_ref, dv_ref, w1_ref, b1_ref, w2_ref, ua_ref, ub_ref):
        dv = dv_ref[...]
        ya = sa_ref[...] * dv
        yb = sb_ref[...] * dv
        h = jnp.dot(ya, w1_ref[:HALF, :], preferred_element_type=jnp.float32)
        h = h + jnp.dot(yb, w1_ref[HALF:, :],
                        preferred_element_type=jnp.float32)
        h = jnp.maximum(h + b1_ref[...], 0.0)
        g = jnp.dot(h, w2_ref[...], preferred_element_type=jnp.float32)
        u2 = g * dv
        ua_ref[...] = u2[:, :HALF]
        ub_ref[...] = u2[:, HALF:]

    grid = (n_pad // blk,)
    return pl.pallas_call(
        body,
        grid=grid,
        in_specs=[
            pl.BlockSpec((blk, HALF), lambda i: (i, 0)),
            pl.BlockSpec((blk, HALF), lambda i: (i, 0)),
            pl.BlockSpec((blk, 1), lambda i: (i, 0)),
            pl.BlockSpec((nin, hid), lambda i: (0, 0)),
            pl.BlockSpec((1, hid), lambda i: (0, 0)),
            pl.BlockSpec((hid, nout), lambda i: (0, 0)),
        ],
        out_specs=[
            pl.BlockSpec((blk, HALF), lambda i: (i, 0)),
            pl.BlockSpec((blk, HALF), lambda i: (i, 0)),
        ],
        out_shape=[
            jax.ShapeDtypeStruct((n_pad, HALF), jnp.float32),
            jax.ShapeDtypeStruct((n_pad, HALF), jnp.float32),
        ],
    )(sa, sb, dinv, W1, b1r, W2)


def _tc_finish(sa, sb, dinv, b2r, n_pad, nout, blk):
    def body(sa_ref, sb_ref, dv_ref, b2_ref, z_ref):
        dv = dv_ref[...]
        z = jnp.concatenate([sa_ref[...] * dv, sb_ref[...] * dv], axis=1)
        z_ref[...] = z + b2_ref[...]

    grid = (n_pad // blk,)
    return pl.pallas_call(
        body,
        grid=grid,
        in_specs=[
            pl.BlockSpec((blk, HALF), lambda i: (i, 0)),
            pl.BlockSpec((blk, HALF), lambda i: (i, 0)),
            pl.BlockSpec((blk, 1), lambda i: (i, 0)),
            pl.BlockSpec((1, nout), lambda i: (0, 0)),
        ],
        out_specs=pl.BlockSpec((blk, nout), lambda i: (i, 0)),
        out_shape=jax.ShapeDtypeStruct((n_pad, nout), jnp.float32),
    )(sa, sb, dinv, b2r)


def kernel(x, edge_index, W1, b1, W2, b2):
    n, nin = x.shape
    hid = W1.shape[1]
    nout = W2.shape[1]
    e = edge_index.shape[1]

    n_pad = ((n + 1023) // 1024) * 1024
    blk = 1024
    x_p = jnp.zeros((n_pad, nin), jnp.float32).at[:n].set(x)

    src = edge_index[0]
    dst = edge_index[1]
    ew = 32 * CH
    e_pad = ((e + ew - 1) // ew) * ew
    if e_pad != e:
        src = jnp.concatenate(
            [src, jnp.zeros((e_pad - e,), jnp.int32)])
        dst = jnp.concatenate(
            [dst, jnp.full((e_pad - e,), n_pad - 1, jnp.int32)])
    src16 = src.reshape(16, -1, CH)
    dst16 = dst.reshape(16, -1, CH)

    zeros_w = jnp.zeros((n_pad, DEGW), jnp.float32)
    ones_w = jnp.ones((CH, DEGW), jnp.float32)

    d0 = _sc_degree(dst16, zeros_w, ones_w, n_pad)

    ua, ub, dinv = _tc_scale_x(x_p, d0, n_pad, nin, blk)
    s1a, s1b = _sc_scatter(ua, ub, src16, dst16, n_pad)
    u2a, u2b = _tc_mlp(s1a, s1b, dinv, W1, b1.reshape(1, hid), W2,
                       n_pad, nin, hid, nout, blk)
    s2a, s2b = _sc_scatter(u2a, u2b, src16, dst16, n_pad)
    z = _tc_finish(s2a, s2b, dinv, b2.reshape(1, nout), n_pad, nout, blk)
    return z[:n]


# speedup vs baseline: 19.7373x; 1.3481x over previous
import functools

import jax
import jax.numpy as jnp
from jax import lax
from jax.experimental import pallas as pl
from jax.experimental.pallas import tpu as pltpu
from jax.experimental.pallas import tpu_sc as plsc

CH = 125
NP = 2
HALF = 128


DEGW = 128


def _sc_degree(dst16, zeros_w, ones_w, n_pad):
    nc = dst16.shape[1]
    rpt = n_pad // 16
    mesh = plsc.VectorSubcoreMesh(core_axis_name="c", subcore_axis_name="s", num_cores=2, num_subcores=16)

    @functools.partial(
        pl.kernel, mesh=mesh,
        out_type=jax.ShapeDtypeStruct((n_pad, DEGW), jnp.float32),
        scratch_types=[
            pltpu.VMEM((nc, CH), jnp.int32),
            pltpu.VMEM((CH, DEGW), jnp.float32),
            pltpu.VMEM_SHARED((n_pad, DEGW), jnp.float32),
        ],
    )
    def deg_kernel(dst_hbm, zero_hbm, one_hbm, out_hbm, idx_v, ones_v, acc):
        c = lax.axis_index("c")
        s = lax.axis_index("s")
        base = s * rpt
        pltpu.sync_copy(dst_hbm.at[s], idx_v)
        pltpu.sync_copy(one_hbm, ones_v)
        pltpu.sync_copy(zero_hbm.at[pl.ds(base, rpt)], acc.at[pl.ds(base, rpt)])
        plsc.subcore_barrier()

        def body(j, carry):
            pltpu.sync_copy(ones_v, acc.at[idx_v.at[j]], add=True)
            return carry

        lax.fori_loop(0, nc, body, 0)
        plsc.subcore_barrier()

        @pl.when(c == 0)
        def _():
            pltpu.sync_copy(acc.at[pl.ds(base, rpt)],
                            out_hbm.at[pl.ds(base, rpt)])

    return deg_kernel(dst16, zeros_w, ones_w)


def _sc_scatter(ua, ub, src16, dst16, n_pad):
    nc = src16.shape[1]
    rpt = n_pad // 16
    mesh = plsc.VectorSubcoreMesh(core_axis_name="c", subcore_axis_name="s", num_cores=2, num_subcores=16)

    @functools.partial(
        pl.kernel, mesh=mesh,
        out_type=[jax.ShapeDtypeStruct((n_pad, HALF), jnp.float32),
                  jax.ShapeDtypeStruct((n_pad, HALF), jnp.float32)],
        scratch_types=[
            pltpu.VMEM((nc // NP, CH), jnp.int32),
            pltpu.VMEM((nc // NP, CH), jnp.int32),
            pltpu.VMEM((CH, HALF), jnp.float32),
            pltpu.VMEM((CH, HALF), jnp.float32),
            pltpu.VMEM_SHARED((n_pad, HALF), jnp.float32),
            pltpu.SemaphoreType.DMA,
            pltpu.SemaphoreType.DMA,
        ],
    )
    def scat_kernel(ua_hbm, ub_hbm, src_hbm, dst_hbm, sa_hbm, sb_hbm,
                    isrc, idst, rows0, rows1, acc, sem0, sem1):
        c = lax.axis_index("c")
        s = lax.axis_index("s")
        base = s * rpt
        nc2 = nc // NP

        def run(u_hbm, out_hbm):
            pltpu.sync_copy(u_hbm.at[pl.ds(base, rpt)],
                            acc.at[pl.ds(base, rpt)])
            plsc.subcore_barrier()

            for p in range(NP):
                pltpu.sync_copy(src_hbm.at[s].at[pl.ds(p * nc2, nc2)], isrc)
                pltpu.sync_copy(dst_hbm.at[s].at[pl.ds(p * nc2, nc2)], idst)

                pltpu.async_copy(u_hbm.at[isrc.at[0]], rows0, sem0)

                def body(j, carry):
                    e0 = 2 * j
                    h1 = pltpu.async_copy(u_hbm.at[isrc.at[e0 + 1]], rows1,
                                          sem1)
                    pltpu.make_async_copy(u_hbm.at[isrc.at[e0]], rows0,
                                          sem0).wait()
                    pltpu.sync_copy(rows0, acc.at[idst.at[e0]], add=True)

                    @pl.when(j < nc2 // 2 - 1)
                    def _():
                        pltpu.async_copy(u_hbm.at[isrc.at[e0 + 2]], rows0,
                                         sem0)

                    h1.wait()
                    pltpu.sync_copy(rows1, acc.at[idst.at[e0 + 1]], add=True)
                    return carry

                lax.fori_loop(0, nc2 // 2, body, 0)

            plsc.subcore_barrier()
            pltpu.sync_copy(acc.at[pl.ds(base, rpt)],
                            out_hbm.at[pl.ds(base, rpt)])

        @pl.when(c == 0)
        def _():
            run(ua_hbm, sa_hbm)

        @pl.when(c == 1)
        def _():
            run(ub_hbm, sb_hbm)

    return scat_kernel(ua, ub, src16, dst16)


def _tc_scale_x(x_p, d0, n_pad, nin, blk):
    def body(x_ref, d0_ref, ua_ref, ub_ref, dv_ref):
        deg = d0_ref[:, :1] + 1.0
        dv = lax.rsqrt(jnp.maximum(deg, 1e-12))
        u = x_ref[...] * dv
        ua_ref[...] = u[:, :HALF]
        ub_ref[...] = u[:, HALF:]
        dv_ref[...] = dv

    grid = (n_pad // blk,)
    return pl.pallas_call(
        body,
        grid=grid,
        in_specs=[
            pl.BlockSpec((blk, nin), lambda i: (i, 0)),
            pl.BlockSpec((blk, DEGW), lambda i: (i, 0)),
        ],
        out_specs=[
            pl.BlockSpec((blk, HALF), lambda i: (i, 0)),
            pl.BlockSpec((blk, HALF), lambda i: (i, 0)),
            pl.BlockSpec((blk, 1), lambda i: (i, 0)),
        ],
        out_shape=[
            jax.ShapeDtypeStruct((n_pad, HALF), jnp.float32),
            jax.ShapeDtypeStruct((n_pad, HALF), jnp.float32),
            jax.ShapeDtypeStruct((n_pad, 1), jnp.float32),
        ],
    )(x_p, d0)


def _tc_mlp(sa, sb, dinv, W1, b1r, W2, n_pad, nin, hid, nout, blk):
    def body(sa_ref, sb_ref, dv_ref, w1_ref, b1_ref, w2_ref, ua_ref, ub_ref):
        dv = dv_ref[...]
        ya = sa_ref[...] * dv
        yb = sb_ref[...] * dv
        h = jnp.dot(ya, w1_ref[:HALF, :], preferred_element_type=jnp.float32)
        h = h + jnp.dot(yb, w1_ref[HALF:, :],
                        preferred_element_type=jnp.float32)
        h = jnp.maximum(h + b1_ref[...], 0.0)
        g = jnp.dot(h, w2_ref[...], preferred_element_type=jnp.float32)
        u2 = g * dv
        ua_ref[...] = u2[:, :HALF]
        ub_ref[...] = u2[:, HALF:]

    grid = (n_pad // blk,)
    return pl.pallas_call(
        body,
        grid=grid,
        in_specs=[
            pl.BlockSpec((blk, HALF), lambda i: (i, 0)),
            pl.BlockSpec((blk, HALF), lambda i: (i, 0)),
            pl.BlockSpec((blk, 1), lambda i: (i, 0)),
            pl.BlockSpec((nin, hid), lambda i: (0, 0)),
            pl.BlockSpec((1, hid), lambda i: (0, 0)),
            pl.BlockSpec((hid, nout), lambda i: (0, 0)),
        ],
        out_specs=[
            pl.BlockSpec((blk, HALF), lambda i: (i, 0)),
            pl.BlockSpec((blk, HALF), lambda i: (i, 0)),
        ],
        out_shape=[
            jax.ShapeDtypeStruct((n_pad, HALF), jnp.float32),
            jax.ShapeDtypeStruct((n_pad, HALF), jnp.float32),
        ],
    )(sa, sb, dinv, W1, b1r, W2)


def _tc_finish(sa, sb, dinv, b2r, n_pad, nout, blk):
    def body(sa_ref, sb_ref, dv_ref, b2_ref, z_ref):
        dv = dv_ref[...]
        z = jnp.concatenate([sa_ref[...] * dv, sb_ref[...] * dv], axis=1)
        z_ref[...] = z + b2_ref[...]

    grid = (n_pad // blk,)
    return pl.pallas_call(
        body,
        grid=grid,
        in_specs=[
            pl.BlockSpec((blk, HALF), lambda i: (i, 0)),
            pl.BlockSpec((blk, HALF), lambda i: (i, 0)),
            pl.BlockSpec((blk, 1), lambda i: (i, 0)),
            pl.BlockSpec((1, nout), lambda i: (0, 0)),
        ],
        out_specs=pl.BlockSpec((blk, nout), lambda i: (i, 0)),
        out_shape=jax.ShapeDtypeStruct((n_pad, nout), jnp.float32),
    )(sa, sb, dinv, b2r)


def kernel(x, edge_index, W1, b1, W2, b2):
    n, nin = x.shape
    hid = W1.shape[1]
    nout = W2.shape[1]
    e = edge_index.shape[1]

    n_pad = ((n + 1023) // 1024) * 1024
    blk = 1024
    x_p = jnp.zeros((n_pad, nin), jnp.float32).at[:n].set(x)

    src = edge_index[0]
    dst = edge_index[1]
    ew = 64 * CH
    e_pad = ((e + ew - 1) // ew) * ew
    if e_pad != e:
        src = jnp.concatenate(
            [src, jnp.zeros((e_pad - e,), jnp.int32)])
        dst = jnp.concatenate(
            [dst, jnp.full((e_pad - e,), n_pad - 1, jnp.int32)])
    src16 = src.reshape(16, -1, CH)
    dst16 = dst.reshape(16, -1, CH)

    zeros_w = jnp.zeros((n_pad, DEGW), jnp.float32)
    ones_w = jnp.ones((CH, DEGW), jnp.float32)

    d0 = _sc_degree(dst16, zeros_w, ones_w, n_pad)

    ua, ub, dinv = _tc_scale_x(x_p, d0, n_pad, nin, blk)
    s1a, s1b = _sc_scatter(ua, ub, src16, dst16, n_pad)
    u2a, u2b = _tc_mlp(s1a, s1b, dinv, W1, b1.reshape(1, hid), W2,
                       n_pad, nin, hid, nout, blk)
    s2a, s2b = _sc_scatter(u2a, u2b, src16, dst16, n_pad)
    z = _tc_finish(s2a, s2b, dinv, b2.reshape(1, nout), n_pad, nout, blk)
    return z[:n]


# speedup vs baseline: 21.0748x; 1.0678x over previous
import functools

import jax
import jax.numpy as jnp
from jax import lax
from jax.experimental import pallas as pl
from jax.experimental.pallas import tpu as pltpu
from jax.experimental.pallas import tpu_sc as plsc

CH = 125
NP = 2
HALF = 128


DEGW = 128


def _sc_degree(dst16, zeros_w, ones_w, n_pad):
    nc = dst16.shape[1]
    ncd = nc // 2
    rpt = n_pad // 16
    mesh = plsc.VectorSubcoreMesh(core_axis_name="c", subcore_axis_name="s", num_cores=2, num_subcores=16)

    @functools.partial(
        pl.kernel, mesh=mesh,
        out_type=[jax.ShapeDtypeStruct((n_pad, DEGW), jnp.float32),
                  jax.ShapeDtypeStruct((n_pad, DEGW), jnp.float32)],
        scratch_types=[
            pltpu.VMEM((ncd, CH), jnp.int32),
            pltpu.VMEM((CH, DEGW), jnp.float32),
            pltpu.VMEM_SHARED((n_pad, DEGW), jnp.float32),
        ],
    )
    def deg_kernel(dst_hbm, zero_hbm, one_hbm, d0_hbm, d1_hbm,
                   idx_v, ones_v, acc):
        c = lax.axis_index("c")
        s = lax.axis_index("s")
        base = s * rpt
        pltpu.sync_copy(one_hbm, ones_v)
        pltpu.sync_copy(zero_hbm.at[pl.ds(base, rpt)], acc.at[pl.ds(base, rpt)])
        plsc.subcore_barrier()

        def run(off, out_hbm):
            pltpu.sync_copy(dst_hbm.at[s].at[pl.ds(off, ncd)], idx_v)

            def body(j, carry):
                pltpu.sync_copy(ones_v, acc.at[idx_v.at[j]], add=True)
                return carry

            lax.fori_loop(0, ncd, body, 0)
            plsc.subcore_barrier()
            pltpu.sync_copy(acc.at[pl.ds(base, rpt)],
                            out_hbm.at[pl.ds(base, rpt)])

        @pl.when(c == 0)
        def _():
            run(0, d0_hbm)

        @pl.when(c == 1)
        def _():
            run(ncd, d1_hbm)

    return deg_kernel(dst16, zeros_w, ones_w)


def _sc_scatter(ua, ub, src16, dst16, n_pad):
    nc = src16.shape[1]
    rpt = n_pad // 16
    mesh = plsc.VectorSubcoreMesh(core_axis_name="c", subcore_axis_name="s", num_cores=2, num_subcores=16)

    @functools.partial(
        pl.kernel, mesh=mesh,
        out_type=[jax.ShapeDtypeStruct((n_pad, HALF), jnp.float32),
                  jax.ShapeDtypeStruct((n_pad, HALF), jnp.float32)],
        scratch_types=[
            pltpu.VMEM((nc // NP, CH), jnp.int32),
            pltpu.VMEM((nc // NP, CH), jnp.int32),
            pltpu.VMEM((CH, HALF), jnp.float32),
            pltpu.VMEM((CH, HALF), jnp.float32),
            pltpu.VMEM_SHARED((n_pad, HALF), jnp.float32),
            pltpu.SemaphoreType.DMA,
            pltpu.SemaphoreType.DMA,
        ],
    )
    def scat_kernel(ua_hbm, ub_hbm, src_hbm, dst_hbm, sa_hbm, sb_hbm,
                    isrc, idst, rows0, rows1, acc, sem0, sem1):
        c = lax.axis_index("c")
        s = lax.axis_index("s")
        base = s * rpt
        nc2 = nc // NP

        def run(u_hbm, out_hbm):
            pltpu.sync_copy(u_hbm.at[pl.ds(base, rpt)],
                            acc.at[pl.ds(base, rpt)])
            plsc.subcore_barrier()

            for p in range(NP):
                pltpu.sync_copy(src_hbm.at[s].at[pl.ds(p * nc2, nc2)], isrc)
                pltpu.sync_copy(dst_hbm.at[s].at[pl.ds(p * nc2, nc2)], idst)

                pltpu.async_copy(u_hbm.at[isrc.at[0]], rows0, sem0)

                def body(j, carry):
                    e0 = 2 * j
                    h1 = pltpu.async_copy(u_hbm.at[isrc.at[e0 + 1]], rows1,
                                          sem1)
                    pltpu.make_async_copy(u_hbm.at[isrc.at[e0]], rows0,
                                          sem0).wait()
                    pltpu.sync_copy(rows0, acc.at[idst.at[e0]], add=True)

                    @pl.when(j < nc2 // 2 - 1)
                    def _():
                        pltpu.async_copy(u_hbm.at[isrc.at[e0 + 2]], rows0,
                                         sem0)

                    h1.wait()
                    pltpu.sync_copy(rows1, acc.at[idst.at[e0 + 1]], add=True)
                    return carry

                lax.fori_loop(0, nc2 // 2, body, 0)

            plsc.subcore_barrier()
            pltpu.sync_copy(acc.at[pl.ds(base, rpt)],
                            out_hbm.at[pl.ds(base, rpt)])

        @pl.when(c == 0)
        def _():
            run(ua_hbm, sa_hbm)

        @pl.when(c == 1)
        def _():
            run(ub_hbm, sb_hbm)

    return scat_kernel(ua, ub, src16, dst16)


def _tc_scale_x(x_p, d0, d1, n_pad, nin, blk):
    def body(x_ref, d0_ref, d1_ref, ua_ref, ub_ref, dv_ref):
        deg = d0_ref[:, :1] + d1_ref[:, :1] + 1.0
        dv = lax.rsqrt(jnp.maximum(deg, 1e-12))
        u = x_ref[...] * dv
        ua_ref[...] = u[:, :HALF]
        ub_ref[...] = u[:, HALF:]
        dv_ref[...] = dv

    grid = (n_pad // blk,)
    return pl.pallas_call(
        body,
        grid=grid,
        in_specs=[
            pl.BlockSpec((blk, nin), lambda i: (i, 0)),
            pl.BlockSpec((blk, DEGW), lambda i: (i, 0)),
            pl.BlockSpec((blk, DEGW), lambda i: (i, 0)),
        ],
        out_specs=[
            pl.BlockSpec((blk, HALF), lambda i: (i, 0)),
            pl.BlockSpec((blk, HALF), lambda i: (i, 0)),
            pl.BlockSpec((blk, 1), lambda i: (i, 0)),
        ],
        out_shape=[
            jax.ShapeDtypeStruct((n_pad, HALF), jnp.float32),
            jax.ShapeDtypeStruct((n_pad, HALF), jnp.float32),
            jax.ShapeDtypeStruct((n_pad, 1), jnp.float32),
        ],
    )(x_p, d0, d1)


def _tc_mlp(sa, sb, dinv, W1, b1r, W2, n_pad, nin, hid, nout, blk):
    def body(sa_ref, sb_ref, dv_ref, w1_ref, b1_ref, w2_ref, ua_ref, ub_ref):
        dv = dv_ref[...]
        ya = sa_ref[...] * dv
        yb = sb_ref[...] * dv
        h = jnp.dot(ya, w1_ref[:HALF, :], preferred_element_type=jnp.float32)
        h = h + jnp.dot(yb, w1_ref[HALF:, :],
                        preferred_element_type=jnp.float32)
        h = jnp.maximum(h + b1_ref[...], 0.0)
        g = jnp.dot(h, w2_ref[...], preferred_element_type=jnp.float32)
        u2 = g * dv
        ua_ref[...] = u2[:, :HALF]
        ub_ref[...] = u2[:, HALF:]

    grid = (n_pad // blk,)
    return pl.pallas_call(
        body,
        grid=grid,
        in_specs=[
            pl.BlockSpec((blk, HALF), lambda i: (i, 0)),
            pl.BlockSpec((blk, HALF), lambda i: (i, 0)),
            pl.BlockSpec((blk, 1), lambda i: (i, 0)),
            pl.BlockSpec((nin, hid), lambda i: (0, 0)),
            pl.BlockSpec((1, hid), lambda i: (0, 0)),
            pl.BlockSpec((hid, nout), lambda i: (0, 0)),
        ],
        out_specs=[
            pl.BlockSpec((blk, HALF), lambda i: (i, 0)),
            pl.BlockSpec((blk, HALF), lambda i: (i, 0)),
        ],
        out_shape=[
            jax.ShapeDtypeStruct((n_pad, HALF), jnp.float32),
            jax.ShapeDtypeStruct((n_pad, HALF), jnp.float32),
        ],
    )(sa, sb, dinv, W1, b1r, W2)


def _tc_finish(sa, sb, dinv, b2r, n_pad, nout, blk):
    def body(sa_ref, sb_ref, dv_ref, b2_ref, z_ref):
        dv = dv_ref[...]
        z = jnp.concatenate([sa_ref[...] * dv, sb_ref[...] * dv], axis=1)
        z_ref[...] = z + b2_ref[...]

    grid = (n_pad // blk,)
    return pl.pallas_call(
        body,
        grid=grid,
        in_specs=[
            pl.BlockSpec((blk, HALF), lambda i: (i, 0)),
            pl.BlockSpec((blk, HALF), lambda i: (i, 0)),
            pl.BlockSpec((blk, 1), lambda i: (i, 0)),
            pl.BlockSpec((1, nout), lambda i: (0, 0)),
        ],
        out_specs=pl.BlockSpec((blk, nout), lambda i: (i, 0)),
        out_shape=jax.ShapeDtypeStruct((n_pad, nout), jnp.float32),
    )(sa, sb, dinv, b2r)


def kernel(x, edge_index, W1, b1, W2, b2):
    n, nin = x.shape
    hid = W1.shape[1]
    nout = W2.shape[1]
    e = edge_index.shape[1]

    n_pad = ((n + 1023) // 1024) * 1024
    blk = 1024
    x_p = jnp.zeros((n_pad, nin), jnp.float32).at[:n].set(x)

    src = edge_index[0]
    dst = edge_index[1]
    ew = 64 * CH
    e_pad = ((e + ew - 1) // ew) * ew
    if e_pad != e:
        src = jnp.concatenate(
            [src, jnp.zeros((e_pad - e,), jnp.int32)])
        dst = jnp.concatenate(
            [dst, jnp.full((e_pad - e,), n_pad - 1, jnp.int32)])
    src16 = src.reshape(16, -1, CH)
    dst16 = dst.reshape(16, -1, CH)

    zeros_w = jnp.zeros((n_pad, DEGW), jnp.float32)
    ones_w = jnp.ones((CH, DEGW), jnp.float32)

    d0, d1 = _sc_degree(dst16, zeros_w, ones_w, n_pad)

    ua, ub, dinv = _tc_scale_x(x_p, d0, d1, n_pad, nin, blk)
    s1a, s1b = _sc_scatter(ua, ub, src16, dst16, n_pad)
    u2a, u2b = _tc_mlp(s1a, s1b, dinv, W1, b1.reshape(1, hid), W2,
                       n_pad, nin, hid, nout, blk)
    s2a, s2b = _sc_scatter(u2a, u2b, src16, dst16, n_pad)
    z = _tc_finish(s2a, s2b, dinv, b2.reshape(1, nout), n_pad, nout, blk)
    return z[:n]


# speedup vs baseline: 22.3047x; 1.0584x over previous
import functools

import jax
import jax.numpy as jnp
from jax import lax
from jax.experimental import pallas as pl
from jax.experimental.pallas import tpu as pltpu
from jax.experimental.pallas import tpu_sc as plsc

CH = 125
NP = 2
HALF = 128


DEGW = 128


def _sc_degree(dst16, zeros_w, ones_w, n_pad):
    nc = dst16.shape[1]
    ncd = nc // 2
    rpt = n_pad // 16
    mesh = plsc.VectorSubcoreMesh(core_axis_name="c", subcore_axis_name="s", num_cores=2, num_subcores=16)

    @functools.partial(
        pl.kernel, mesh=mesh,
        out_type=[jax.ShapeDtypeStruct((n_pad, DEGW), jnp.float32),
                  jax.ShapeDtypeStruct((n_pad, DEGW), jnp.float32)],
        scratch_types=[
            pltpu.VMEM((ncd, CH), jnp.int32),
            pltpu.VMEM((CH, DEGW), jnp.float32),
            pltpu.VMEM_SHARED((n_pad, DEGW), jnp.float32),
        ],
    )
    def deg_kernel(dst_hbm, zero_hbm, one_hbm, d0_hbm, d1_hbm,
                   idx_v, ones_v, acc):
        c = lax.axis_index("c")
        s = lax.axis_index("s")
        base = s * rpt
        pltpu.sync_copy(one_hbm, ones_v)
        pltpu.sync_copy(zero_hbm.at[pl.ds(base, rpt)], acc.at[pl.ds(base, rpt)])
        plsc.subcore_barrier()

        def run(off, out_hbm):
            pltpu.sync_copy(dst_hbm.at[s].at[pl.ds(off, ncd)], idx_v)

            def body(j, carry):
                pltpu.sync_copy(ones_v, acc.at[idx_v.at[j]], add=True)
                return carry

            lax.fori_loop(0, ncd, body, 0)
            plsc.subcore_barrier()
            pltpu.sync_copy(acc.at[pl.ds(base, rpt)],
                            out_hbm.at[pl.ds(base, rpt)])

        @pl.when(c == 0)
        def _():
            run(0, d0_hbm)

        @pl.when(c == 1)
        def _():
            run(ncd, d1_hbm)

    return deg_kernel(dst16, zeros_w, ones_w)


def _sc_scatter(ua, ub, src16, dst16, n_pad):
    nc = src16.shape[1]
    rpt = n_pad // 16
    mesh = plsc.VectorSubcoreMesh(core_axis_name="c", subcore_axis_name="s", num_cores=2, num_subcores=16)

    @functools.partial(
        pl.kernel, mesh=mesh,
        out_type=[jax.ShapeDtypeStruct((n_pad, HALF), jnp.float32),
                  jax.ShapeDtypeStruct((n_pad, HALF), jnp.float32)],
        scratch_types=[
            pltpu.VMEM((nc // NP, CH), jnp.int32),
            pltpu.VMEM((nc // NP, CH), jnp.int32),
            pltpu.VMEM((CH, HALF), jnp.float32),
            pltpu.VMEM((CH, HALF), jnp.float32),
            pltpu.VMEM_SHARED((n_pad, HALF), jnp.float32),
            pltpu.SemaphoreType.DMA,
            pltpu.SemaphoreType.DMA,
        ],
    )
    def scat_kernel(ua_hbm, ub_hbm, src_hbm, dst_hbm, sa_hbm, sb_hbm,
                    isrc, idst, rows0, rows1, acc, sem0, sem1):
        c = lax.axis_index("c")
        s = lax.axis_index("s")
        base = s * rpt
        nc2 = nc // NP

        def run(u_hbm, out_hbm):
            pltpu.sync_copy(u_hbm.at[pl.ds(base, rpt)],
                            acc.at[pl.ds(base, rpt)])
            plsc.subcore_barrier()

            for p in range(NP):
                pltpu.sync_copy(src_hbm.at[s].at[pl.ds(p * nc2, nc2)], isrc)
                pltpu.sync_copy(dst_hbm.at[s].at[pl.ds(p * nc2, nc2)], idst)

                pltpu.async_copy(u_hbm.at[isrc.at[0]], rows0, sem0)

                def body(j, carry):
                    e0 = 2 * j
                    h1 = pltpu.async_copy(u_hbm.at[isrc.at[e0 + 1]], rows1,
                                          sem1)
                    pltpu.make_async_copy(u_hbm.at[isrc.at[e0]], rows0,
                                          sem0).wait()
                    pltpu.sync_copy(rows0, acc.at[idst.at[e0]], add=True)

                    @pl.when(j < nc2 // 2 - 1)
                    def _():
                        pltpu.async_copy(u_hbm.at[isrc.at[e0 + 2]], rows0,
                                         sem0)

                    h1.wait()
                    pltpu.sync_copy(rows1, acc.at[idst.at[e0 + 1]], add=True)
                    return carry

                lax.fori_loop(0, nc2 // 2, body, 0)

            plsc.subcore_barrier()
            pltpu.sync_copy(acc.at[pl.ds(base, rpt)],
                            out_hbm.at[pl.ds(base, rpt)])

        @pl.when(c == 0)
        def _():
            run(ua_hbm, sa_hbm)

        @pl.when(c == 1)
        def _():
            run(ub_hbm, sb_hbm)

    return scat_kernel(ua, ub, src16, dst16)


def _tc_scale_x(x_p, d0, d1, n, n_pad, nin, blk):
    def body(x_ref, d0_ref, d1_ref, ua_ref, ub_ref, dv_ref):
        deg = d0_ref[:, :1] + d1_ref[:, :1] + 1.0
        dv = lax.rsqrt(jnp.maximum(deg, 1e-12))
        u = x_ref[...] * dv
        ua_ref[...] = u[:, :HALF]
        ub_ref[...] = u[:, HALF:]
        dv_ref[...] = dv

    grid = (n // blk,)
    return pl.pallas_call(
        body,
        grid=grid,
        in_specs=[
            pl.BlockSpec((blk, nin), lambda i: (i, 0)),
            pl.BlockSpec((blk, DEGW), lambda i: (i, 0)),
            pl.BlockSpec((blk, DEGW), lambda i: (i, 0)),
        ],
        out_specs=[
            pl.BlockSpec((blk, HALF), lambda i: (i, 0)),
            pl.BlockSpec((blk, HALF), lambda i: (i, 0)),
            pl.BlockSpec((blk, 1), lambda i: (i, 0)),
        ],
        out_shape=[
            jax.ShapeDtypeStruct((n_pad, HALF), jnp.float32),
            jax.ShapeDtypeStruct((n_pad, HALF), jnp.float32),
            jax.ShapeDtypeStruct((n_pad, 1), jnp.float32),
        ],
    )(x_p, d0, d1)


def _tc_mlp(sa, sb, dinv, W1, b1r, W2, n, n_pad, nin, hid, nout, blk):
    def body(sa_ref, sb_ref, dv_ref, w1_ref, b1_ref, w2_ref, ua_ref, ub_ref):
        dv = dv_ref[...]
        ya = sa_ref[...] * dv
        yb = sb_ref[...] * dv
        h = jnp.dot(ya, w1_ref[:HALF, :], preferred_element_type=jnp.float32)
        h = h + jnp.dot(yb, w1_ref[HALF:, :],
                        preferred_element_type=jnp.float32)
        h = jnp.maximum(h + b1_ref[...], 0.0)
        g = jnp.dot(h, w2_ref[...], preferred_element_type=jnp.float32)
        u2 = g * dv
        ua_ref[...] = u2[:, :HALF]
        ub_ref[...] = u2[:, HALF:]

    grid = (n // blk,)
    return pl.pallas_call(
        body,
        grid=grid,
        in_specs=[
            pl.BlockSpec((blk, HALF), lambda i: (i, 0)),
            pl.BlockSpec((blk, HALF), lambda i: (i, 0)),
            pl.BlockSpec((blk, 1), lambda i: (i, 0)),
            pl.BlockSpec((nin, hid), lambda i: (0, 0)),
            pl.BlockSpec((1, hid), lambda i: (0, 0)),
            pl.BlockSpec((hid, nout), lambda i: (0, 0)),
        ],
        out_specs=[
            pl.BlockSpec((blk, HALF), lambda i: (i, 0)),
            pl.BlockSpec((blk, HALF), lambda i: (i, 0)),
        ],
        out_shape=[
            jax.ShapeDtypeStruct((n_pad, HALF), jnp.float32),
            jax.ShapeDtypeStruct((n_pad, HALF), jnp.float32),
        ],
    )(sa, sb, dinv, W1, b1r, W2)


def _tc_finish(sa, sb, dinv, b2r, n, nout, blk):
    def body(sa_ref, sb_ref, dv_ref, b2_ref, z_ref):
        dv = dv_ref[...]
        z = jnp.concatenate([sa_ref[...] * dv, sb_ref[...] * dv], axis=1)
        z_ref[...] = z + b2_ref[...]

    grid = (n // blk,)
    return pl.pallas_call(
        body,
        grid=grid,
        in_specs=[
            pl.BlockSpec((blk, HALF), lambda i: (i, 0)),
            pl.BlockSpec((blk, HALF), lambda i: (i, 0)),
            pl.BlockSpec((blk, 1), lambda i: (i, 0)),
            pl.BlockSpec((1, nout), lambda i: (0, 0)),
        ],
        out_specs=pl.BlockSpec((blk, nout), lambda i: (i, 0)),
        out_shape=jax.ShapeDtypeStruct((n, nout), jnp.float32),
    )(sa, sb, dinv, b2r)


def kernel(x, edge_index, W1, b1, W2, b2):
    n, nin = x.shape
    hid = W1.shape[1]
    nout = W2.shape[1]
    e = edge_index.shape[1]

    n_pad = ((n + 1023) // 1024) * 1024
    blk = 1
    for cand in (2048, 2000, 1024, 1000, 512, 500, 256, 200, 128, 100, 8):
        if n % cand == 0 and cand % 8 == 0:
            blk = cand
            break
    assert blk > 1, "n has no row-block divisor that is a multiple of 8"

    src = edge_index[0]
    dst = edge_index[1]
    ew = 64 * CH
    e_pad = ((e + ew - 1) // ew) * ew
    if e_pad != e:
        src = jnp.concatenate(
            [src, jnp.zeros((e_pad - e,), jnp.int32)])
        dst = jnp.concatenate(
            [dst, jnp.full((e_pad - e,), n_pad - 1, jnp.int32)])
    src16 = src.reshape(16, -1, CH)
    dst16 = dst.reshape(16, -1, CH)

    zeros_w = jnp.zeros((n_pad, DEGW), jnp.float32)
    ones_w = jnp.ones((CH, DEGW), jnp.float32)

    d0, d1 = _sc_degree(dst16, zeros_w, ones_w, n_pad)

    ua, ub, dinv = _tc_scale_x(x, d0, d1, n, n_pad, nin, blk)
    s1a, s1b = _sc_scatter(ua, ub, src16, dst16, n_pad)
    u2a, u2b = _tc_mlp(s1a, s1b, dinv, W1, b1.reshape(1, hid), W2,
                       n, n_pad, nin, hid, nout, blk)
    s2a, s2b = _sc_scatter(u2a, u2b, src16, dst16, n_pad)
    return _tc_finish(s2a, s2b, dinv, b2.reshape(1, nout), n, nout, blk)
